# baseline (speedup 1.0000x reference)
import jax
import jax.numpy as jnp
from jax.experimental import pallas as pl
from jax.experimental.pallas import tpu as pltpu

_LANES = 512


def _ciou_block(px_ref, py_ref, out_ref):
    px = px_ref[...]
    py = py_ref[...]

    ax, bx = px[0:8, :], px[8:16, :]
    ay, by = py[0:8, :], py[8:16, :]

    def nxt(v):
        return jnp.concatenate([v[1:8, :], v[0:1, :]], axis=0)

    dax, day = nxt(ax) - ax, nxt(ay) - ay
    dbx, dby = nxt(bx) - bx, nxt(by) - by

    area_a = 0.5 * jnp.sum(ax * day - ay * dax, axis=0, keepdims=True)
    area_b = 0.5 * jnp.sum(bx * dby - by * dbx, axis=0, keepdims=True)

    def clip(sx, sy, dx, dy, hx, hy, hdx, hdy):
        t1 = jnp.zeros_like(sx)
        t2 = jnp.ones_like(sx)
        dead = jnp.zeros(sx.shape, dtype=jnp.bool_)
        for j in range(8):
            ex = hdx[j:j + 1, :]
            ey = hdy[j:j + 1, :]
            c0 = ex * (sy - hy[j:j + 1, :]) - ey * (sx - hx[j:j + 1, :])
            cd = ex * dy - ey * dx
            para = cd == 0.0
            r = -c0 / jnp.where(para, 1.0, cd)
            t1 = jnp.maximum(t1, jnp.where(cd > 0.0, r, 0.0))
            t2 = jnp.minimum(t2, jnp.where(cd < 0.0, r, 1.0))
            dead = dead | (para & (c0 < 0.0))
        keep = (t2 > t1) & ~dead
        return jnp.where(keep, (t2 - t1) * (sx * dy - sy * dx), 0.0)

    inter = 0.5 * (
        jnp.sum(clip(ax, ay, dax, day, bx, by, dbx, dby), axis=0, keepdims=True)
        + jnp.sum(clip(bx, by, dbx, dby, ax, ay, dax, day), axis=0,
                  keepdims=True))

    rows = jax.lax.broadcasted_iota(jnp.int32, px.shape, 0)
    acc = jnp.zeros_like(area_a)
    for i in range(16):
        pix = px[i:i + 1, :]
        piy = py[i:i + 1, :]
        ex = px - pix
        ey = py - piy
        mincr = jnp.zeros_like(px)
        for k in range(16):
            if k == i:
                continue
            wx = px[k:k + 1, :] - pix
            wy = py[k:k + 1, :] - piy
            cr = jnp.where(rows == k, 0.0, ex * wy - ey * wx)
            mincr = jnp.minimum(mincr, cr)
        contrib = jnp.where(mincr >= 0.0, pix * py - piy * px, 0.0)
        acc = acc + jnp.sum(contrib, axis=0, keepdims=True)
    ch_area = 0.5 * acc

    union = area_a + area_b - inter
    iou = inter / union
    out_ref[...] = iou - (ch_area - union) / ch_area


def kernel(a, b):
    bsz = a.shape[0]
    pts = jnp.concatenate([a, b], axis=1)
    px = pts[..., 0].T
    py = pts[..., 1].T
    g = bsz // _LANES
    ciou = pl.pallas_call(
        _ciou_block,
        grid=(g,),
        in_specs=[pl.BlockSpec((16, _LANES), lambda i: (0, i)),
                  pl.BlockSpec((16, _LANES), lambda i: (0, i))],
        out_specs=pl.BlockSpec((1, _LANES), lambda i: (0, i)),
        out_shape=jax.ShapeDtypeStruct((1, bsz), jnp.float32),
        compiler_params=pltpu.CompilerParams(
            dimension_semantics=("parallel",)),
        name="ciou_mean",
    )(px, py)
    return jnp.mean(ciou)


# speedup vs baseline: 2.3892x; 2.3892x over previous
import jax
import jax.numpy as jnp
from jax.experimental import pallas as pl
from jax.experimental.pallas import tpu as pltpu

_LANES = 512


def _ciou_block(px_ref, py_ref, out_ref):
    px = px_ref[...]
    py = py_ref[...]

    ax, bx = px[0:8, :], px[8:16, :]
    ay, by = py[0:8, :], py[8:16, :]

    def nxt(v):
        return jnp.concatenate([v[1:8, :], v[0:1, :]], axis=0)

    dax, day = nxt(ax) - ax, nxt(ay) - ay
    dbx, dby = nxt(bx) - bx, nxt(by) - by

    area_a = 0.5 * jnp.sum(ax * day - ay * dax, axis=0, keepdims=True)
    area_b = 0.5 * jnp.sum(bx * dby - by * dbx, axis=0, keepdims=True)

    def clip(sx, sy, dx, dy, hx, hy, hdx, hdy):
        t1 = jnp.zeros_like(sx)
        t2 = jnp.ones_like(sx)
        dead = jnp.zeros(sx.shape, dtype=jnp.bool_)
        for j in range(8):
            ex = hdx[j:j + 1, :]
            ey = hdy[j:j + 1, :]
            c0 = ex * (sy - hy[j:j + 1, :]) - ey * (sx - hx[j:j + 1, :])
            cd = ex * dy - ey * dx
            para = cd == 0.0
            r = -c0 / jnp.where(para, 1.0, cd)
            t1 = jnp.maximum(t1, jnp.where(cd > 0.0, r, 0.0))
            t2 = jnp.minimum(t2, jnp.where(cd < 0.0, r, 1.0))
            dead = dead | (para & (c0 < 0.0))
        keep = (t2 > t1) & ~dead
        return jnp.where(keep, (t2 - t1) * (sx * dy - sy * dx), 0.0)

    inter = 0.5 * (
        jnp.sum(clip(ax, ay, dax, day, bx, by, dbx, dby), axis=0, keepdims=True)
        + jnp.sum(clip(bx, by, dbx, dby, ax, ay, dax, day), axis=0,
                  keepdims=True))

    nax, nay = nxt(ax), nxt(ay)
    nbx, nby = nxt(bx), nxt(by)

    def prv(v):
        return jnp.concatenate([v[7:8, :], v[0:7, :]], axis=0)

    pbx, pby = prv(bx), prv(by)
    pax, pay = prv(ax), prv(ay)

    acc8 = jnp.zeros_like(ax)

    mincr = None
    for k in range(8):
        wx = bx[k:k + 1, :] - ax
        wy = by[k:k + 1, :] - ay
        cr = dax * wy - day * wx
        mincr = cr if mincr is None else jnp.minimum(mincr, cr)
    acc8 = acc8 + jnp.where(mincr >= 0.0, ax * nay - ay * nax, 0.0)

    mincr = None
    for k in range(8):
        wx = ax[k:k + 1, :] - bx
        wy = ay[k:k + 1, :] - by
        cr = dbx * wy - dby * wx
        mincr = cr if mincr is None else jnp.minimum(mincr, cr)
    acc8 = acc8 + jnp.where(mincr >= 0.0, bx * nby - by * nbx, 0.0)

    def bridges(ox, oy, dox, doy, pox, poy, qx, qy, nqx, nqy, pqx, pqy):
        out = jnp.zeros_like(qx)
        for i in range(8):
            oxi = ox[i:i + 1, :]
            oyi = oy[i:i + 1, :]
            vx = qx - oxi
            vy = qy - oyi
            cr1 = vx * doy[i:i + 1, :] - vy * dox[i:i + 1, :]
            cr2 = vy * (oxi - pox[i:i + 1, :]) - vx * (oyi - poy[i:i + 1, :])
            cr3 = vx * (pqy - oyi) - vy * (pqx - oxi)
            cr4 = vx * (nqy - oyi) - vy * (nqx - oxi)
            m = jnp.minimum(jnp.minimum(cr1, cr2), jnp.minimum(cr3, cr4))
            out = out + jnp.where(m >= 0.0, oxi * qy - oyi * qx, 0.0)
        return out

    acc8 = acc8 + bridges(ax, ay, dax, day, pax, pay,
                          bx, by, nbx, nby, pbx, pby)
    acc8 = acc8 + bridges(bx, by, dbx, dby, pbx, pby,
                          ax, ay, nax, nay, pax, pay)
    ch_area = 0.5 * jnp.sum(acc8, axis=0, keepdims=True)

    union = area_a + area_b - inter
    iou = inter / union
    out_ref[...] = iou - (ch_area - union) / ch_area


def kernel(a, b):
    bsz = a.shape[0]
    pts = jnp.concatenate([a, b], axis=1)
    px = pts[..., 0].T
    py = pts[..., 1].T
    g = bsz // _LANES
    ciou = pl.pallas_call(
        _ciou_block,
        grid=(g,),
        in_specs=[pl.BlockSpec((16, _LANES), lambda i: (0, i)),
                  pl.BlockSpec((16, _LANES), lambda i: (0, i))],
        out_specs=pl.BlockSpec((1, _LANES), lambda i: (0, i)),
        out_shape=jax.ShapeDtypeStruct((1, bsz), jnp.float32),
        compiler_params=pltpu.CompilerParams(
            dimension_semantics=("parallel",)),
        name="ciou_mean",
    )(px, py)
    return jnp.mean(ciou)


# speedup vs baseline: 2.4690x; 1.0334x over previous
import jax
import jax.numpy as jnp
from jax.experimental import pallas as pl
from jax.experimental.pallas import tpu as pltpu

_LANES = 512


def _ciou_block(px_ref, py_ref, out_ref):
    px = px_ref[...]
    py = py_ref[...]

    ax, bx = px[0:8, :], px[8:16, :]
    ay, by = py[0:8, :], py[8:16, :]

    def nxt(v):
        return jnp.concatenate([v[1:8, :], v[0:1, :]], axis=0)

    dax, day = nxt(ax) - ax, nxt(ay) - ay
    dbx, dby = nxt(bx) - bx, nxt(by) - by

    ab_area = 0.5 * jnp.sum(
        (ax * day - ay * dax) + (bx * dby - by * dbx), axis=0, keepdims=True)

    def clip(sx, sy, dx, dy, hx, hy, hdx, hdy):
        t1 = t2 = None
        for j in range(8):
            ex = hdx[j:j + 1, :]
            ey = hdy[j:j + 1, :]
            c0 = ex * (sy - hy[j:j + 1, :]) - ey * (sx - hx[j:j + 1, :])
            cd = ex * dy - ey * dx
            r = -c0 / cd
            lo = jnp.where(cd > 0.0, r, 0.0)
            hi = jnp.where(cd < 0.0, r, 1.0)
            t1 = lo if t1 is None else jnp.maximum(t1, lo)
            t2 = hi if t2 is None else jnp.minimum(t2, hi)
        return jnp.where(t2 > t1, (t2 - t1) * (sx * dy - sy * dx), 0.0)

    inter = 0.5 * jnp.sum(
        clip(ax, ay, dax, day, bx, by, dbx, dby)
        + clip(bx, by, dbx, dby, ax, ay, dax, day), axis=0, keepdims=True)

    nax, nay = nxt(ax), nxt(ay)
    nbx, nby = nxt(bx), nxt(by)

    def prv(v):
        return jnp.concatenate([v[7:8, :], v[0:7, :]], axis=0)

    pbx, pby = prv(bx), prv(by)
    pax, pay = prv(ax), prv(ay)

    acc8 = jnp.zeros_like(ax)

    mincr = None
    for k in range(8):
        wx = bx[k:k + 1, :] - ax
        wy = by[k:k + 1, :] - ay
        cr = dax * wy - day * wx
        mincr = cr if mincr is None else jnp.minimum(mincr, cr)
    acc8 = acc8 + jnp.where(mincr >= 0.0, ax * nay - ay * nax, 0.0)

    mincr = None
    for k in range(8):
        wx = ax[k:k + 1, :] - bx
        wy = ay[k:k + 1, :] - by
        cr = dbx * wy - dby * wx
        mincr = cr if mincr is None else jnp.minimum(mincr, cr)
    acc8 = acc8 + jnp.where(mincr >= 0.0, bx * nby - by * nbx, 0.0)

    def bridges(ox, oy, dox, doy, pox, poy, qx, qy, nqx, nqy, pqx, pqy):
        out = jnp.zeros_like(qx)
        for i in range(8):
            oxi = ox[i:i + 1, :]
            oyi = oy[i:i + 1, :]
            vx = qx - oxi
            vy = qy - oyi
            cr1 = vx * doy[i:i + 1, :] - vy * dox[i:i + 1, :]
            cr2 = vy * (oxi - pox[i:i + 1, :]) - vx * (oyi - poy[i:i + 1, :])
            cr3 = vx * (pqy - oyi) - vy * (pqx - oxi)
            cr4 = vx * (nqy - oyi) - vy * (nqx - oxi)
            m = jnp.minimum(jnp.minimum(cr1, cr2), jnp.minimum(cr3, cr4))
            out = out + jnp.where(m >= 0.0, oxi * qy - oyi * qx, 0.0)
        return out

    acc8 = acc8 + bridges(ax, ay, dax, day, pax, pay,
                          bx, by, nbx, nby, pbx, pby)
    acc8 = acc8 + bridges(bx, by, dbx, dby, pbx, pby,
                          ax, ay, nax, nay, pax, pay)
    ch_area = 0.5 * jnp.sum(acc8, axis=0, keepdims=True)

    union = ab_area - inter
    iou = inter / union
    out_ref[...] = iou - (ch_area - union) / ch_area


def kernel(a, b):
    bsz = a.shape[0]
    pts = jnp.concatenate([a, b], axis=1)
    px = pts[..., 0].T
    py = pts[..., 1].T
    g = bsz // _LANES
    ciou = pl.pallas_call(
        _ciou_block,
        grid=(g,),
        in_specs=[pl.BlockSpec((16, _LANES), lambda i: (0, i)),
                  pl.BlockSpec((16, _LANES), lambda i: (0, i))],
        out_specs=pl.BlockSpec((1, _LANES), lambda i: (0, i)),
        out_shape=jax.ShapeDtypeStruct((1, bsz), jnp.float32),
        compiler_params=pltpu.CompilerParams(
            dimension_semantics=("parallel",)),
        name="ciou_mean",
    )(px, py)
    return jnp.mean(ciou)


# speedup vs baseline: 3.0389x; 1.2308x over previous
import jax
import jax.numpy as jnp
from jax.experimental import pallas as pl
from jax.experimental.pallas import tpu as pltpu

_LANES = 1024


def _ciou_block(px_ref, py_ref, out_ref):
    px = px_ref[...]
    py = py_ref[...]

    ax, bx = px[0:8, :], px[8:16, :]
    ay, by = py[0:8, :], py[8:16, :]

    def nxt(v):
        return jnp.concatenate([v[1:8, :], v[0:1, :]], axis=0)

    dax, day = nxt(ax) - ax, nxt(ay) - ay
    dbx, dby = nxt(bx) - bx, nxt(by) - by

    ab_area = 0.5 * jnp.sum(
        (ax * day - ay * dax) + (bx * dby - by * dbx), axis=0, keepdims=True)

    def clip(sx, sy, dx, dy, hx, hy, hdx, hdy):
        t1 = t2 = None
        for j in range(8):
            ex = hdx[j:j + 1, :]
            ey = hdy[j:j + 1, :]
            c0 = ex * (sy - hy[j:j + 1, :]) - ey * (sx - hx[j:j + 1, :])
            cd = ex * dy - ey * dx
            r = -c0 / cd
            lo = jnp.where(cd > 0.0, r, 0.0)
            hi = jnp.where(cd < 0.0, r, 1.0)
            t1 = lo if t1 is None else jnp.maximum(t1, lo)
            t2 = hi if t2 is None else jnp.minimum(t2, hi)
        return jnp.where(t2 > t1, (t2 - t1) * (sx * dy - sy * dx), 0.0)

    inter = 0.5 * jnp.sum(
        clip(ax, ay, dax, day, bx, by, dbx, dby)
        + clip(bx, by, dbx, dby, ax, ay, dax, day), axis=0, keepdims=True)

    nax, nay = nxt(ax), nxt(ay)
    nbx, nby = nxt(bx), nxt(by)

    def prv(v):
        return jnp.concatenate([v[7:8, :], v[0:7, :]], axis=0)

    pbx, pby = prv(bx), prv(by)
    pax, pay = prv(ax), prv(ay)

    acc8 = jnp.zeros_like(ax)

    mincr = None
    for k in range(8):
        wx = bx[k:k + 1, :] - ax
        wy = by[k:k + 1, :] - ay
        cr = dax * wy - day * wx
        mincr = cr if mincr is None else jnp.minimum(mincr, cr)
    acc8 = acc8 + jnp.where(mincr >= 0.0, ax * nay - ay * nax, 0.0)

    mincr = None
    for k in range(8):
        wx = ax[k:k + 1, :] - bx
        wy = ay[k:k + 1, :] - by
        cr = dbx * wy - dby * wx
        mincr = cr if mincr is None else jnp.minimum(mincr, cr)
    acc8 = acc8 + jnp.where(mincr >= 0.0, bx * nby - by * nbx, 0.0)

    def bridges(ox, oy, dox, doy, pox, poy, qx, qy, nqx, nqy, pqx, pqy):
        out = jnp.zeros_like(qx)
        for i in range(8):
            oxi = ox[i:i + 1, :]
            oyi = oy[i:i + 1, :]
            vx = qx - oxi
            vy = qy - oyi
            cr1 = vx * doy[i:i + 1, :] - vy * dox[i:i + 1, :]
            cr2 = vy * (oxi - pox[i:i + 1, :]) - vx * (oyi - poy[i:i + 1, :])
            cr3 = vx * (pqy - oyi) - vy * (pqx - oxi)
            cr4 = vx * (nqy - oyi) - vy * (nqx - oxi)
            m = jnp.minimum(jnp.minimum(cr1, cr2), jnp.minimum(cr3, cr4))
            out = out + jnp.where(m >= 0.0, oxi * qy - oyi * qx, 0.0)
        return out

    acc8 = acc8 + bridges(ax, ay, dax, day, pax, pay,
                          bx, by, nbx, nby, pbx, pby)
    acc8 = acc8 + bridges(bx, by, dbx, dby, pbx, pby,
                          ax, ay, nax, nay, pax, pay)
    ch_area = 0.5 * jnp.sum(acc8, axis=0, keepdims=True)

    union = ab_area - inter
    iou = inter / union
    out_ref[...] = iou - (ch_area - union) / ch_area


def kernel(a, b):
    bsz = a.shape[0]
    pts = jnp.concatenate([a, b], axis=1)
    px = pts[..., 0].T
    py = pts[..., 1].T
    g = bsz // _LANES
    ciou = pl.pallas_call(
        _ciou_block,
        grid=(g,),
        in_specs=[pl.BlockSpec((16, _LANES), lambda i: (0, i)),
                  pl.BlockSpec((16, _LANES), lambda i: (0, i))],
        out_specs=pl.BlockSpec((1, _LANES), lambda i: (0, i)),
        out_shape=jax.ShapeDtypeStruct((1, bsz), jnp.float32),
        compiler_params=pltpu.CompilerParams(
            dimension_semantics=("parallel",)),
        name="ciou_mean",
    )(px, py)
    return jnp.mean(ciou)


# speedup vs baseline: 3.0983x; 1.0195x over previous
import jax
import jax.numpy as jnp
from jax.experimental import pallas as pl
from jax.experimental.pallas import tpu as pltpu

_LANES = 4096


def _ciou_block(px_ref, py_ref, out_ref):
    px = px_ref[...]
    py = py_ref[...]

    ax, bx = px[0:8, :], px[8:16, :]
    ay, by = py[0:8, :], py[8:16, :]

    def nxt(v):
        return jnp.concatenate([v[1:8, :], v[0:1, :]], axis=0)

    dax, day = nxt(ax) - ax, nxt(ay) - ay
    dbx, dby = nxt(bx) - bx, nxt(by) - by

    ab_area = 0.5 * jnp.sum(
        (ax * day - ay * dax) + (bx * dby - by * dbx), axis=0, keepdims=True)

    def clip(sx, sy, dx, dy, hx, hy, hdx, hdy):
        t1 = t2 = None
        for j in range(8):
            ex = hdx[j:j + 1, :]
            ey = hdy[j:j + 1, :]
            c0 = ex * (sy - hy[j:j + 1, :]) - ey * (sx - hx[j:j + 1, :])
            cd = ex * dy - ey * dx
            r = -c0 / cd
            lo = jnp.where(cd > 0.0, r, 0.0)
            hi = jnp.where(cd < 0.0, r, 1.0)
            t1 = lo if t1 is None else jnp.maximum(t1, lo)
            t2 = hi if t2 is None else jnp.minimum(t2, hi)
        return jnp.where(t2 > t1, (t2 - t1) * (sx * dy - sy * dx), 0.0)

    inter = 0.5 * jnp.sum(
        clip(ax, ay, dax, day, bx, by, dbx, dby)
        + clip(bx, by, dbx, dby, ax, ay, dax, day), axis=0, keepdims=True)

    nax, nay = nxt(ax), nxt(ay)
    nbx, nby = nxt(bx), nxt(by)

    def prv(v):
        return jnp.concatenate([v[7:8, :], v[0:7, :]], axis=0)

    pbx, pby = prv(bx), prv(by)
    pax, pay = prv(ax), prv(ay)

    acc8 = jnp.zeros_like(ax)

    mincr = None
    for k in range(8):
        wx = bx[k:k + 1, :] - ax
        wy = by[k:k + 1, :] - ay
        cr = dax * wy - day * wx
        mincr = cr if mincr is None else jnp.minimum(mincr, cr)
    acc8 = acc8 + jnp.where(mincr >= 0.0, ax * nay - ay * nax, 0.0)

    mincr = None
    for k in range(8):
        wx = ax[k:k + 1, :] - bx
        wy = ay[k:k + 1, :] - by
        cr = dbx * wy - dby * wx
        mincr = cr if mincr is None else jnp.minimum(mincr, cr)
    acc8 = acc8 + jnp.where(mincr >= 0.0, bx * nby - by * nbx, 0.0)

    def bridges(ox, oy, dox, doy, pox, poy, qx, qy, nqx, nqy, pqx, pqy):
        out = jnp.zeros_like(qx)
        for i in range(8):
            oxi = ox[i:i + 1, :]
            oyi = oy[i:i + 1, :]
            vx = qx - oxi
            vy = qy - oyi
            cr1 = vx * doy[i:i + 1, :] - vy * dox[i:i + 1, :]
            cr2 = vy * (oxi - pox[i:i + 1, :]) - vx * (oyi - poy[i:i + 1, :])
            cr3 = vx * (pqy - oyi) - vy * (pqx - oxi)
            cr4 = vx * (nqy - oyi) - vy * (nqx - oxi)
            m = jnp.minimum(jnp.minimum(cr1, cr2), jnp.minimum(cr3, cr4))
            out = out + jnp.where(m >= 0.0, oxi * qy - oyi * qx, 0.0)
        return out

    acc8 = acc8 + bridges(ax, ay, dax, day, pax, pay,
                          bx, by, nbx, nby, pbx, pby)
    acc8 = acc8 + bridges(bx, by, dbx, dby, pbx, pby,
                          ax, ay, nax, nay, pax, pay)
    ch_area = 0.5 * jnp.sum(acc8, axis=0, keepdims=True)

    union = ab_area - inter
    iou = inter / union
    out_ref[...] = iou - (ch_area - union) / ch_area


def kernel(a, b):
    bsz = a.shape[0]
    pts = jnp.concatenate([a, b], axis=1)
    px = pts[..., 0].T
    py = pts[..., 1].T
    g = bsz // _LANES
    ciou = pl.pallas_call(
        _ciou_block,
        grid=(g,),
        in_specs=[pl.BlockSpec((16, _LANES), lambda i: (0, i)),
                  pl.BlockSpec((16, _LANES), lambda i: (0, i))],
        out_specs=pl.BlockSpec((1, _LANES), lambda i: (0, i)),
        out_shape=jax.ShapeDtypeStruct((1, bsz), jnp.float32),
        compiler_params=pltpu.CompilerParams(
            dimension_semantics=("parallel",)),
        name="ciou_mean",
    )(px, py)
    return jnp.mean(ciou)


# speedup vs baseline: 3.3347x; 1.0763x over previous
import jax
import jax.numpy as jnp
from jax.experimental import pallas as pl
from jax.experimental.pallas import tpu as pltpu

_LANES = 4096


def _ciou_block(px_ref, py_ref, out_ref):
    px = px_ref[...]
    py = py_ref[...]

    ax, bx = px[0:8, :], px[8:16, :]
    ay, by = py[0:8, :], py[8:16, :]

    def nxt(v):
        return jnp.concatenate([v[1:8, :], v[0:1, :]], axis=0)

    dax, day = nxt(ax) - ax, nxt(ay) - ay
    dbx, dby = nxt(bx) - bx, nxt(by) - by

    cada = ax * day - ay * dax
    cbdb = bx * dby - by * dbx
    ab_area = 0.5 * jnp.sum(cada + cbdb, axis=0, keepdims=True)

    def clip(sx, sy, dx, dy, csd, hdx, hdy, chd):
        t1 = t2 = None
        for j in range(8):
            ex = hdx[j:j + 1, :]
            ey = hdy[j:j + 1, :]
            c0 = ex * sy - ey * sx + chd[j:j + 1, :]
            cd = ex * dy - ey * dx
            r = -c0 / cd
            lo = jnp.where(cd > 0.0, r, 0.0)
            hi = jnp.where(cd < 0.0, r, 1.0)
            t1 = lo if t1 is None else jnp.maximum(t1, lo)
            t2 = hi if t2 is None else jnp.minimum(t2, hi)
        return jnp.where(t2 > t1, (t2 - t1) * csd, 0.0)

    inter = 0.5 * jnp.sum(
        clip(ax, ay, dax, day, cada, dbx, dby, cbdb)
        + clip(bx, by, dbx, dby, cbdb, dax, day, cada), axis=0, keepdims=True)

    def prv(v):
        return jnp.concatenate([v[7:8, :], v[0:7, :]], axis=0)

    pdax, pday = prv(dax), prv(day)
    pdbx, pdby = prv(dbx), prv(dby)

    mincr = None
    for k in range(8):
        cr = dax * by[k:k + 1, :] - day * bx[k:k + 1, :]
        mincr = cr if mincr is None else jnp.minimum(mincr, cr)
    acc8 = jnp.where(mincr + cada >= 0.0, cada, 0.0)

    mincr = None
    for k in range(8):
        cr = dbx * ay[k:k + 1, :] - dby * ax[k:k + 1, :]
        mincr = cr if mincr is None else jnp.minimum(mincr, cr)
    acc8 = acc8 + jnp.where(mincr + cbdb >= 0.0, cbdb, 0.0)

    def bridges(ox, oy, dox, doy, pdox, pdoy, qx, qy, dqx, dqy, pdqx, pdqy):
        out = None
        for i in range(8):
            oxi = ox[i:i + 1, :]
            oyi = oy[i:i + 1, :]
            vx = qx - oxi
            vy = qy - oyi
            cr1 = vx * doy[i:i + 1, :] - vy * dox[i:i + 1, :]
            cr2 = vy * pdox[i:i + 1, :] - vx * pdoy[i:i + 1, :]
            cr3 = vy * pdqx - vx * pdqy
            cr4 = vx * dqy - vy * dqx
            m = jnp.minimum(jnp.minimum(cr1, cr2), jnp.minimum(cr3, cr4))
            c = jnp.where(m >= 0.0, oxi * qy - oyi * qx, 0.0)
            out = c if out is None else out + c
        return out

    acc8 = acc8 + bridges(ax, ay, dax, day, pdax, pday,
                          bx, by, dbx, dby, pdbx, pdby)
    acc8 = acc8 + bridges(bx, by, dbx, dby, pdbx, pdby,
                          ax, ay, dax, day, pdax, pday)
    ch_area = 0.5 * jnp.sum(acc8, axis=0, keepdims=True)

    union = ab_area - inter
    iou = inter / union
    out_ref[...] = iou - (ch_area - union) / ch_area


def kernel(a, b):
    bsz = a.shape[0]
    pts = jnp.concatenate([a, b], axis=1)
    px = pts[..., 0].T
    py = pts[..., 1].T
    g = bsz // _LANES
    ciou = pl.pallas_call(
        _ciou_block,
        grid=(g,),
        in_specs=[pl.BlockSpec((16, _LANES), lambda i: (0, i)),
                  pl.BlockSpec((16, _LANES), lambda i: (0, i))],
        out_specs=pl.BlockSpec((1, _LANES), lambda i: (0, i)),
        out_shape=jax.ShapeDtypeStruct((1, bsz), jnp.float32),
        compiler_params=pltpu.CompilerParams(
            dimension_semantics=("parallel",)),
        name="ciou_mean",
    )(px, py)
    return jnp.mean(ciou)


# speedup vs baseline: 3.3381x; 1.0010x over previous
import jax
import jax.numpy as jnp
from jax.experimental import pallas as pl
from jax.experimental.pallas import tpu as pltpu

_LANES = 8192


def _ciou_block(px_ref, py_ref, out_ref):
    px = px_ref[...]
    py = py_ref[...]

    ax, bx = px[0:8, :], px[8:16, :]
    ay, by = py[0:8, :], py[8:16, :]

    def nxt(v):
        return jnp.concatenate([v[1:8, :], v[0:1, :]], axis=0)

    dax, day = nxt(ax) - ax, nxt(ay) - ay
    dbx, dby = nxt(bx) - bx, nxt(by) - by

    cada = ax * day - ay * dax
    cbdb = bx * dby - by * dbx
    ab_area = 0.5 * jnp.sum(cada + cbdb, axis=0, keepdims=True)

    def clip(sx, sy, dx, dy, csd, hdx, hdy, chd):
        t1 = t2 = None
        for j in range(8):
            ex = hdx[j:j + 1, :]
            ey = hdy[j:j + 1, :]
            c0 = ex * sy - ey * sx + chd[j:j + 1, :]
            cd = ex * dy - ey * dx
            r = -c0 / cd
            lo = jnp.where(cd > 0.0, r, 0.0)
            hi = jnp.where(cd < 0.0, r, 1.0)
            t1 = lo if t1 is None else jnp.maximum(t1, lo)
            t2 = hi if t2 is None else jnp.minimum(t2, hi)
        return jnp.where(t2 > t1, (t2 - t1) * csd, 0.0)

    inter = 0.5 * jnp.sum(
        clip(ax, ay, dax, day, cada, dbx, dby, cbdb)
        + clip(bx, by, dbx, dby, cbdb, dax, day, cada), axis=0, keepdims=True)

    def prv(v):
        return jnp.concatenate([v[7:8, :], v[0:7, :]], axis=0)

    pdax, pday = prv(dax), prv(day)
    pdbx, pdby = prv(dbx), prv(dby)

    mincr = None
    for k in range(8):
        cr = dax * by[k:k + 1, :] - day * bx[k:k + 1, :]
        mincr = cr if mincr is None else jnp.minimum(mincr, cr)
    acc8 = jnp.where(mincr + cada >= 0.0, cada, 0.0)

    mincr = None
    for k in range(8):
        cr = dbx * ay[k:k + 1, :] - dby * ax[k:k + 1, :]
        mincr = cr if mincr is None else jnp.minimum(mincr, cr)
    acc8 = acc8 + jnp.where(mincr + cbdb >= 0.0, cbdb, 0.0)

    def bridges(ox, oy, dox, doy, pdox, pdoy, qx, qy, dqx, dqy, pdqx, pdqy):
        out = None
        for i in range(8):
            oxi = ox[i:i + 1, :]
            oyi = oy[i:i + 1, :]
            vx = qx - oxi
            vy = qy - oyi
            cr1 = vx * doy[i:i + 1, :] - vy * dox[i:i + 1, :]
            cr2 = vy * pdox[i:i + 1, :] - vx * pdoy[i:i + 1, :]
            cr3 = vy * pdqx - vx * pdqy
            cr4 = vx * dqy - vy * dqx
            m = jnp.minimum(jnp.minimum(cr1, cr2), jnp.minimum(cr3, cr4))
            c = jnp.where(m >= 0.0, oxi * qy - oyi * qx, 0.0)
            out = c if out is None else out + c
        return out

    acc8 = acc8 + bridges(ax, ay, dax, day, pdax, pday,
                          bx, by, dbx, dby, pdbx, pdby)
    acc8 = acc8 + bridges(bx, by, dbx, dby, pdbx, pdby,
                          ax, ay, dax, day, pdax, pday)
    ch_area = 0.5 * jnp.sum(acc8, axis=0, keepdims=True)

    union = ab_area - inter
    iou = inter / union
    out_ref[...] = iou - (ch_area - union) / ch_area


def kernel(a, b):
    bsz = a.shape[0]
    pts = jnp.concatenate([a, b], axis=1)
    px = pts[..., 0].T
    py = pts[..., 1].T
    g = bsz // _LANES
    ciou = pl.pallas_call(
        _ciou_block,
        grid=(g,),
        in_specs=[pl.BlockSpec((16, _LANES), lambda i: (0, i)),
                  pl.BlockSpec((16, _LANES), lambda i: (0, i))],
        out_specs=pl.BlockSpec((1, _LANES), lambda i: (0, i)),
        out_shape=jax.ShapeDtypeStruct((1, bsz), jnp.float32),
        compiler_params=pltpu.CompilerParams(
            dimension_semantics=("parallel",)),
        name="ciou_mean",
    )(px, py)
    return jnp.mean(ciou)
